# Initial kernel scaffold; baseline (speedup 1.0000x reference)
import jax
import jax.numpy as jnp
from jax.experimental import pallas as pl


def kernel(imgs, ctx_conv0_c1_w, ctx_conv0_c1_b, ctx_conv0_c1_a, ctx_conv0_c2_w, ctx_conv0_c2_b, ctx_conv0_c2_a, ctx_conv1_c1_w, ctx_conv1_c1_b, ctx_conv1_c1_a, ctx_conv1_c2_w, ctx_conv1_c2_b, ctx_conv1_c2_a, ctx_conv2_c1_w, ctx_conv2_c1_b, ctx_conv2_c1_a, ctx_conv2_c2_w, ctx_conv2_c2_b, ctx_conv2_c2_a, ctx_conv3_c1_w, ctx_conv3_c1_b, ctx_conv3_c1_a, ctx_conv3_c2_w, ctx_conv3_c2_b, ctx_conv3_c2_a, ctx_conv4_c1_w, ctx_conv4_c1_b, ctx_conv4_c1_a, ctx_conv4_c2_w, ctx_conv4_c2_b, ctx_conv4_c2_a, fus_conv0_c1_w, fus_conv0_c1_b, fus_conv0_c1_a, fus_conv0_c2_w, fus_conv0_c2_b, fus_conv0_c2_a, fus_down0_c1_w, fus_down0_c1_b, fus_down0_c1_a, fus_down0_c2_w, fus_down0_c2_b, fus_down0_c2_a, fus_down1_c1_w, fus_down1_c1_b, fus_down1_c1_a, fus_down1_c2_w, fus_down1_c2_b, fus_down1_c2_a, fus_down2_c1_w, fus_down2_c1_b, fus_down2_c1_a, fus_down2_c2_w, fus_down2_c2_b, fus_down2_c2_a, fus_down3_c1_w, fus_down3_c1_b, fus_down3_c1_a, fus_down3_c2_w, fus_down3_c2_b, fus_down3_c2_a, fus_up0_w, fus_up0_b, fus_up0_a, fus_up1_w, fus_up1_b, fus_up1_a, fus_up2_w, fus_up2_b, fus_up2_a, fus_up3_w, fus_up3_b, fus_up3_a, fus_conv_w, fus_conv_b, fus_conv_a):
    raise NotImplementedError("write your pallas kernel here")


# speedup vs baseline: 1.1282x; 1.1282x over previous
import jax
import jax.numpy as jnp
from jax.experimental import pallas as pl
from jax.experimental.pallas import tpu as pltpu


def _mm_prelu_kernel(x_ref, w_ref, b_ref, a_ref, o_ref):
    acc = jnp.dot(x_ref[...], w_ref[...], preferred_element_type=jnp.float32)
    y = acc + b_ref[...]
    o_ref[...] = jnp.where(y >= 0.0, y, y * a_ref[...]).astype(o_ref.dtype)


def _tile_m(m):
    if m >= 8192:
        return 512
    if m >= 512:
        return 256
    return 128


def _mm_prelu(x, w, b, a, out_dtype=jnp.bfloat16):
    m, k = x.shape
    n = w.shape[1]
    tm = _tile_m(m)
    mp = -(-m // tm) * tm
    if mp != m:
        x = jnp.pad(x, ((0, mp - m), (0, 0)))
    out = pl.pallas_call(
        _mm_prelu_kernel,
        out_shape=jax.ShapeDtypeStruct((mp, n), out_dtype),
        grid=(mp // tm,),
        in_specs=[
            pl.BlockSpec((tm, k), lambda i: (i, 0)),
            pl.BlockSpec((k, n), lambda i: (0, 0)),
            pl.BlockSpec((1, n), lambda i: (0, 0)),
            pl.BlockSpec((1, n), lambda i: (0, 0)),
        ],
        out_specs=pl.BlockSpec((tm, n), lambda i: (i, 0)),
        compiler_params=pltpu.CompilerParams(
            dimension_semantics=("parallel",),
            vmem_limit_bytes=60 * 1024 * 1024,
        ),
    )(x, w, b.astype(jnp.float32).reshape(1, n), a.astype(jnp.float32).reshape(1, n))
    return out[:m]


def _patches(x, kk, stride):
    n, h, w, c = x.shape
    xp = jnp.pad(x, ((0, 0), (1, 1), (1, 1), (0, 0)))
    ho = (h + 2 - kk) // stride + 1
    wo = (w + 2 - kk) // stride + 1
    taps = [xp[:, i:i + stride * ho:stride, j:j + stride * wo:stride, :]
            for i in range(kk) for j in range(kk)]
    cols = jnp.concatenate(taps, axis=-1)
    return cols.reshape(n * ho * wo, kk * kk * c), n, ho, wo


def _conv3x3(x, w, b, a, stride):
    cout, cin, kh, kw = w.shape
    cols, n, ho, wo = _patches(x.astype(jnp.bfloat16), kh, stride)
    wm = jnp.transpose(w, (2, 3, 1, 0)).reshape(kh * kw * cin, cout)
    y = _mm_prelu(cols, wm.astype(jnp.bfloat16), b, a)
    return y.reshape(n, ho, wo, cout)


def _block(x, p, stride=2):
    x = _conv3x3(x, p[0], p[1], p[2], stride)
    return _conv3x3(x, p[3], p[4], p[5], 1)


def _deconv_weights(w):
    cin, cout = w.shape[0], w.shape[1]
    big = jnp.zeros((3, 3, cin, 2, 2, cout), w.dtype)
    for py in range(2):
        for r in range(2):
            for px in range(2):
                for s in range(2):
                    big = big.at[py + r, px + s, :, py, px, :].set(
                        w[:, :, 3 - py - 2 * r, 3 - px - 2 * s])
    return big.reshape(9 * cin, 4 * cout)


def _depth_to_space(y, n, h, w, cout):
    y = y.reshape(n, h, w, 2, 2, cout)
    return jnp.transpose(y, (0, 1, 3, 2, 4, 5)).reshape(n, 2 * h, 2 * w, cout)


def _deconv(x, w, b, a):
    cout = w.shape[1]
    cols, n, h, wd = _patches(x.astype(jnp.bfloat16), 3, 1)
    wm = _deconv_weights(w).astype(jnp.bfloat16)
    y = _mm_prelu(cols, wm, jnp.tile(b, 4), jnp.tile(a, 4))
    return _depth_to_space(y, n, h, wd, cout)


def _final_kernel(x_ref, w_ref, b_ref, bmat_ref, w0_ref, w1_ref, o_ref):
    acc = jnp.dot(x_ref[...], w_ref[...], preferred_element_type=jnp.float32)
    refine = (acc + b_ref[...]).astype(jnp.bfloat16).astype(jnp.float32)
    s = jax.nn.sigmoid(refine)
    mask = jnp.dot(s, bmat_ref[...], preferred_element_type=jnp.float32)
    w0 = w0_ref[...].astype(jnp.float32)
    w1 = w1_ref[...].astype(jnp.float32)
    merged = w0 * mask + w1 * (1.0 - mask) + (s * 2.0 - 1.0)
    o_ref[...] = jnp.clip(merged, 0.0, 1.0)


def _final_deconv_blend(x, w, b, w0, w1):
    cols, n, h, wd = _patches(x.astype(jnp.bfloat16), 3, 1)
    wm = _deconv_weights(w).astype(jnp.bfloat16)
    bb = jnp.tile(b, 4).astype(jnp.float32).reshape(1, 16)
    def s2d(img):
        v = img.reshape(n, h, 2, wd, 2, 3).transpose(0, 1, 3, 2, 4, 5)
        v = jnp.pad(v, ((0, 0),) * 5 + ((0, 1),))
        return v.reshape(n * h * wd, 16)
    bmat = jnp.zeros((16, 16), jnp.float32)
    for p in range(4):
        bmat = bmat.at[4 * p + 3, 4 * p:4 * p + 3].set(1.0)

    m = cols.shape[0]
    tm = _tile_m(m)
    out = pl.pallas_call(
        _final_kernel,
        out_shape=jax.ShapeDtypeStruct((m, 16), jnp.float32),
        grid=(m // tm,),
        in_specs=[
            pl.BlockSpec((tm, cols.shape[1]), lambda i: (i, 0)),
            pl.BlockSpec((cols.shape[1], 16), lambda i: (0, 0)),
            pl.BlockSpec((1, 16), lambda i: (0, 0)),
            pl.BlockSpec((16, 16), lambda i: (0, 0)),
            pl.BlockSpec((tm, 16), lambda i: (i, 0)),
            pl.BlockSpec((tm, 16), lambda i: (i, 0)),
        ],
        out_specs=pl.BlockSpec((tm, 16), lambda i: (i, 0)),
        compiler_params=pltpu.CompilerParams(
            dimension_semantics=("parallel",),
            vmem_limit_bytes=60 * 1024 * 1024,
        ),
    )(cols, wm, bb, bmat, s2d(w0), s2d(w1))
    out = out.reshape(n, h, wd, 2, 2, 4)[..., :3]
    out = out.transpose(0, 1, 3, 2, 4, 5).reshape(n, 2 * h, 2 * wd, 3)
    return jnp.transpose(out, (0, 3, 1, 2))


def _warp(x, flow):
    n, h, w, c = x.shape
    gy, gx = jnp.meshgrid(jnp.arange(h, dtype=jnp.float32),
                          jnp.arange(w, dtype=jnp.float32), indexing='ij')
    sx = jnp.clip(gx[None] + flow[..., 0], 0.0, w - 1.0)
    sy = jnp.clip(gy[None] + flow[..., 1], 0.0, h - 1.0)
    x0 = jnp.floor(sx)
    y0 = jnp.floor(sy)
    wx = (sx - x0)[..., None]
    wy = (sy - y0)[..., None]
    x0i = jnp.clip(x0.astype(jnp.int32), 0, w - 1)
    y0i = jnp.clip(y0.astype(jnp.int32), 0, h - 1)
    x1i = jnp.minimum(x0i + 1, w - 1)
    y1i = jnp.minimum(y0i + 1, h - 1)
    bidx = jnp.arange(n)[:, None, None]
    v00 = x[bidx, y0i, x0i]
    v01 = x[bidx, y0i, x1i]
    v10 = x[bidx, y1i, x0i]
    v11 = x[bidx, y1i, x1i]
    out = (v00 * (1 - wx) * (1 - wy) + v01 * wx * (1 - wy)
           + v10 * (1 - wx) * wy + v11 * wx * wy)
    return out.astype(x.dtype)


def _resize_half(x):
    return _resize(x, x.shape[1] // 2, x.shape[2] // 2)


def _resize(x, oh, ow):
    n, h, w, c = x.shape
    sy = jnp.maximum((jnp.arange(oh, dtype=jnp.float32) + 0.5) * (h / oh) - 0.5, 0.0)
    sx = jnp.maximum((jnp.arange(ow, dtype=jnp.float32) + 0.5) * (w / ow) - 0.5, 0.0)
    y0 = jnp.clip(jnp.floor(sy).astype(jnp.int32), 0, h - 1)
    x0 = jnp.clip(jnp.floor(sx).astype(jnp.int32), 0, w - 1)
    y1 = jnp.minimum(y0 + 1, h - 1)
    x1 = jnp.minimum(x0 + 1, w - 1)
    wy = (sy - y0.astype(jnp.float32))[None, :, None, None]
    wx = (sx - x0.astype(jnp.float32))[None, None, :, None]
    r = x[:, y0] * (1 - wy) + x[:, y1] * wy
    return (r[:, :, x0] * (1 - wx) + r[:, :, x1] * wx).astype(x.dtype)


def kernel(imgs, ctx_conv0_c1_w, ctx_conv0_c1_b, ctx_conv0_c1_a, ctx_conv0_c2_w, ctx_conv0_c2_b, ctx_conv0_c2_a, ctx_conv1_c1_w, ctx_conv1_c1_b, ctx_conv1_c1_a, ctx_conv1_c2_w, ctx_conv1_c2_b, ctx_conv1_c2_a, ctx_conv2_c1_w, ctx_conv2_c1_b, ctx_conv2_c1_a, ctx_conv2_c2_w, ctx_conv2_c2_b, ctx_conv2_c2_a, ctx_conv3_c1_w, ctx_conv3_c1_b, ctx_conv3_c1_a, ctx_conv3_c2_w, ctx_conv3_c2_b, ctx_conv3_c2_a, ctx_conv4_c1_w, ctx_conv4_c1_b, ctx_conv4_c1_a, ctx_conv4_c2_w, ctx_conv4_c2_b, ctx_conv4_c2_a, fus_conv0_c1_w, fus_conv0_c1_b, fus_conv0_c1_a, fus_conv0_c2_w, fus_conv0_c2_b, fus_conv0_c2_a, fus_down0_c1_w, fus_down0_c1_b, fus_down0_c1_a, fus_down0_c2_w, fus_down0_c2_b, fus_down0_c2_a, fus_down1_c1_w, fus_down1_c1_b, fus_down1_c1_a, fus_down1_c2_w, fus_down1_c2_b, fus_down1_c2_a, fus_down2_c1_w, fus_down2_c1_b, fus_down2_c1_a, fus_down2_c2_w, fus_down2_c2_b, fus_down2_c2_a, fus_down3_c1_w, fus_down3_c1_b, fus_down3_c1_a, fus_down3_c2_w, fus_down3_c2_b, fus_down3_c2_a, fus_up0_w, fus_up0_b, fus_up0_a, fus_up1_w, fus_up1_b, fus_up1_a, fus_up2_w, fus_up2_b, fus_up2_a, fus_up3_w, fus_up3_b, fus_up3_a, fus_conv_w, fus_conv_b, fus_conv_a):
    n, _, h, w = imgs.shape

    pooled = imgs.reshape(n, 6, h // 2, 2, w // 2, 2).mean(axis=(3, 5))
    f = jnp.stack([pooled[:, 0] - pooled[:, 3], pooled[:, 1] - pooled[:, 4],
                   pooled[:, 3] - pooled[:, 0], pooled[:, 4] - pooled[:, 1]], axis=1)
    flow = jnp.transpose(jnp.tanh(f) * 2.0, (0, 2, 3, 1))

    img0 = jnp.transpose(imgs[:, :3], (0, 2, 3, 1)).astype(jnp.bfloat16)
    img1 = jnp.transpose(imgs[:, 3:], (0, 2, 3, 1)).astype(jnp.bfloat16)

    ctx = [
        (ctx_conv0_c1_w, ctx_conv0_c1_b, ctx_conv0_c1_a,
         ctx_conv0_c2_w, ctx_conv0_c2_b, ctx_conv0_c2_a),
        (ctx_conv1_c1_w, ctx_conv1_c1_b, ctx_conv1_c1_a,
         ctx_conv1_c2_w, ctx_conv1_c2_b, ctx_conv1_c2_a),
        (ctx_conv2_c1_w, ctx_conv2_c1_b, ctx_conv2_c1_a,
         ctx_conv2_c2_w, ctx_conv2_c2_b, ctx_conv2_c2_a),
        (ctx_conv3_c1_w, ctx_conv3_c1_b, ctx_conv3_c1_a,
         ctx_conv3_c2_w, ctx_conv3_c2_b, ctx_conv3_c2_a),
        (ctx_conv4_c1_w, ctx_conv4_c1_b, ctx_conv4_c1_a,
         ctx_conv4_c2_w, ctx_conv4_c2_b, ctx_conv4_c2_a),
    ]
    xb = jnp.concatenate([img0, img1], axis=0)
    fb = jnp.concatenate([flow[..., :2], flow[..., 2:4]], axis=0)
    xb = _block(xb, ctx[0])
    xb = _block(xb, ctx[1])
    feats = []
    for lvl in range(2, 5):
        fb = _resize_half(fb) * 0.5
        feats.append(_warp(xb, fb))
        xb = _block(xb, ctx[lvl])
    fb = _resize_half(fb) * 0.5
    feats.append(_warp(xb, fb))
    c0 = [fz[:n] for fz in feats]
    c1 = [fz[n:] for fz in feats]

    flow_up = _resize(flow, h, w) * 2.0
    w0 = _warp(img0, flow_up[..., :2])
    w1 = _warp(img1, flow_up[..., 2:4])
    x = jnp.concatenate([w0, w1, flow_up.astype(jnp.bfloat16)], axis=-1)
    x = _block(x, (fus_conv0_c1_w, fus_conv0_c1_b, fus_conv0_c1_a,
                   fus_conv0_c2_w, fus_conv0_c2_b, fus_conv0_c2_a))
    s0 = _block(x, (fus_down0_c1_w, fus_down0_c1_b, fus_down0_c1_a,
                    fus_down0_c2_w, fus_down0_c2_b, fus_down0_c2_a))
    s1 = _block(jnp.concatenate([s0, c0[0], c1[0]], -1),
                (fus_down1_c1_w, fus_down1_c1_b, fus_down1_c1_a,
                 fus_down1_c2_w, fus_down1_c2_b, fus_down1_c2_a))
    s2 = _block(jnp.concatenate([s1, c0[1], c1[1]], -1),
                (fus_down2_c1_w, fus_down2_c1_b, fus_down2_c1_a,
                 fus_down2_c2_w, fus_down2_c2_b, fus_down2_c2_a))
    s3 = _block(jnp.concatenate([s2, c0[2], c1[2]], -1),
                (fus_down3_c1_w, fus_down3_c1_b, fus_down3_c1_a,
                 fus_down3_c2_w, fus_down3_c2_b, fus_down3_c2_a))
    x = _deconv(jnp.concatenate([s3, c0[3], c1[3]], -1), fus_up0_w, fus_up0_b, fus_up0_a)
    x = _deconv(jnp.concatenate([x, s2], -1), fus_up1_w, fus_up1_b, fus_up1_a)
    x = _deconv(jnp.concatenate([x, s1], -1), fus_up2_w, fus_up2_b, fus_up2_a)
    x = _deconv(jnp.concatenate([x, s0], -1), fus_up3_w, fus_up3_b, fus_up3_a)
    return _final_deconv_blend(x, fus_conv_w, fus_conv_b, w0, w1)


# speedup vs baseline: 3.5354x; 3.1336x over previous
import jax
import jax.numpy as jnp
from jax.experimental import pallas as pl
from jax.experimental.pallas import tpu as pltpu


def _mm_prelu_kernel(x_ref, w_ref, b_ref, a_ref, o_ref):
    acc = jnp.dot(x_ref[...], w_ref[...], preferred_element_type=jnp.float32)
    y = acc + b_ref[...]
    o_ref[...] = jnp.where(y >= 0.0, y, y * a_ref[...]).astype(o_ref.dtype)


def _tile_m(m):
    if m >= 8192:
        return 512
    if m >= 512:
        return 256
    return 128


def _mm_prelu(x, w, b, a, out_dtype=jnp.bfloat16):
    m, k = x.shape
    n = w.shape[1]
    tm = _tile_m(m)
    mp = -(-m // tm) * tm
    if mp != m:
        x = jnp.pad(x, ((0, mp - m), (0, 0)))
    out = pl.pallas_call(
        _mm_prelu_kernel,
        out_shape=jax.ShapeDtypeStruct((mp, n), out_dtype),
        grid=(mp // tm,),
        in_specs=[
            pl.BlockSpec((tm, k), lambda i: (i, 0)),
            pl.BlockSpec((k, n), lambda i: (0, 0)),
            pl.BlockSpec((1, n), lambda i: (0, 0)),
            pl.BlockSpec((1, n), lambda i: (0, 0)),
        ],
        out_specs=pl.BlockSpec((tm, n), lambda i: (i, 0)),
        compiler_params=pltpu.CompilerParams(
            dimension_semantics=("parallel",),
            vmem_limit_bytes=60 * 1024 * 1024,
        ),
    )(x, w, b.astype(jnp.float32).reshape(1, n), a.astype(jnp.float32).reshape(1, n))
    return out[:m]


def _patches(x, kk, stride):
    n, h, w, c = x.shape
    xp = jnp.pad(x, ((0, 0), (1, 1), (1, 1), (0, 0)))
    ho = (h + 2 - kk) // stride + 1
    wo = (w + 2 - kk) // stride + 1
    taps = [xp[:, i:i + stride * ho:stride, j:j + stride * wo:stride, :]
            for i in range(kk) for j in range(kk)]
    cols = jnp.concatenate(taps, axis=-1)
    return cols.reshape(n * ho * wo, kk * kk * c), n, ho, wo


def _conv3x3(x, w, b, a, stride):
    cout, cin, kh, kw = w.shape
    cols, n, ho, wo = _patches(x.astype(jnp.bfloat16), kh, stride)
    wm = jnp.transpose(w, (2, 3, 1, 0)).reshape(kh * kw * cin, cout)
    y = _mm_prelu(cols, wm.astype(jnp.bfloat16), b, a)
    return y.reshape(n, ho, wo, cout)


def _block(x, p, stride=2):
    x = _conv3x3(x, p[0], p[1], p[2], stride)
    return _conv3x3(x, p[3], p[4], p[5], 1)


def _deconv_weights(w):
    cin, cout = w.shape[0], w.shape[1]
    big = jnp.zeros((3, 3, cin, 2, 2, cout), w.dtype)
    for py in range(2):
        for r in range(2):
            for px in range(2):
                for s in range(2):
                    big = big.at[py + r, px + s, :, py, px, :].set(
                        w[:, :, 3 - py - 2 * r, 3 - px - 2 * s])
    return big.reshape(9 * cin, 4 * cout)


def _depth_to_space(y, n, h, w, cout):
    y = y.reshape(n, h, w, 2, 2, cout)
    return jnp.transpose(y, (0, 1, 3, 2, 4, 5)).reshape(n, 2 * h, 2 * w, cout)


def _deconv(x, w, b, a):
    cout = w.shape[1]
    cols, n, h, wd = _patches(x.astype(jnp.bfloat16), 3, 1)
    wm = _deconv_weights(w).astype(jnp.bfloat16)
    y = _mm_prelu(cols, wm, jnp.tile(b, 4), jnp.tile(a, 4))
    return _depth_to_space(y, n, h, wd, cout)


def _final_kernel(x_ref, w_ref, b_ref, bmat_ref, w0_ref, w1_ref, o_ref):
    acc = jnp.dot(x_ref[...], w_ref[...], preferred_element_type=jnp.float32)
    refine = (acc + b_ref[...]).astype(jnp.bfloat16).astype(jnp.float32)
    s = jax.nn.sigmoid(refine)
    mask = jnp.dot(s, bmat_ref[...], preferred_element_type=jnp.float32)
    w0 = w0_ref[...].astype(jnp.float32)
    w1 = w1_ref[...].astype(jnp.float32)
    merged = w0 * mask + w1 * (1.0 - mask) + (s * 2.0 - 1.0)
    o_ref[...] = jnp.clip(merged, 0.0, 1.0)


def _final_deconv_blend(x, w, b, w0, w1):
    cols, n, h, wd = _patches(x.astype(jnp.bfloat16), 3, 1)
    wm = _deconv_weights(w).astype(jnp.bfloat16)
    bb = jnp.tile(b, 4).astype(jnp.float32).reshape(1, 16)
    def s2d(img):
        v = img.reshape(n, h, 2, wd, 2, 3).transpose(0, 1, 3, 2, 4, 5)
        v = jnp.pad(v, ((0, 0),) * 5 + ((0, 1),))
        return v.reshape(n * h * wd, 16)
    bmat = jnp.zeros((16, 16), jnp.float32)
    for p in range(4):
        bmat = bmat.at[4 * p + 3, 4 * p:4 * p + 3].set(1.0)

    m = cols.shape[0]
    tm = _tile_m(m)
    out = pl.pallas_call(
        _final_kernel,
        out_shape=jax.ShapeDtypeStruct((m, 16), jnp.float32),
        grid=(m // tm,),
        in_specs=[
            pl.BlockSpec((tm, cols.shape[1]), lambda i: (i, 0)),
            pl.BlockSpec((cols.shape[1], 16), lambda i: (0, 0)),
            pl.BlockSpec((1, 16), lambda i: (0, 0)),
            pl.BlockSpec((16, 16), lambda i: (0, 0)),
            pl.BlockSpec((tm, 16), lambda i: (i, 0)),
            pl.BlockSpec((tm, 16), lambda i: (i, 0)),
        ],
        out_specs=pl.BlockSpec((tm, 16), lambda i: (i, 0)),
        compiler_params=pltpu.CompilerParams(
            dimension_semantics=("parallel",),
            vmem_limit_bytes=60 * 1024 * 1024,
        ),
    )(cols, wm, bb, bmat, s2d(w0), s2d(w1))
    out = out.reshape(n, h, wd, 2, 2, 4)[..., :3]
    out = out.transpose(0, 1, 3, 2, 4, 5).reshape(n, 2 * h, 2 * wd, 3)
    return jnp.transpose(out, (0, 3, 1, 2))


def _warp(x, flow, radius):
    n, h, w, c = x.shape
    r = radius
    gy = jnp.arange(h, dtype=jnp.float32)[None, :, None]
    gx = jnp.arange(w, dtype=jnp.float32)[None, None, :]
    sx = jnp.clip(gx + flow[..., 0], 0.0, w - 1.0)
    sy = jnp.clip(gy + flow[..., 1], 0.0, h - 1.0)
    x0 = jnp.floor(sx)
    y0 = jnp.floor(sy)
    wx = sx - x0
    wy = sy - y0
    x0i = x0.astype(jnp.int32)
    y0i = y0.astype(jnp.int32)
    x1i = jnp.minimum(x0i + 1, w - 1)
    y1i = jnp.minimum(y0i + 1, h - 1)
    dx0 = x0i - gx.astype(jnp.int32)
    dx1 = x1i - gx.astype(jnp.int32)
    dy0 = y0i - gy.astype(jnp.int32)
    dy1 = y1i - gy.astype(jnp.int32)
    wxd = [jnp.where(dx0 == d, 1.0 - wx, 0.0) + jnp.where(dx1 == d, wx, 0.0)
           for d in range(-r, r + 1)]
    wyd = [jnp.where(dy0 == d, 1.0 - wy, 0.0) + jnp.where(dy1 == d, wy, 0.0)
           for d in range(-r, r + 1)]
    xp = jnp.pad(x, ((0, 0), (r, r), (r, r), (0, 0)))
    out = jnp.zeros((n, h, w, c), jnp.float32)
    for iy, dy in enumerate(range(-r, r + 1)):
        for ix, dx in enumerate(range(-r, r + 1)):
            tap = xp[:, r + dy:r + dy + h, r + dx:r + dx + w, :]
            out = out + (wyd[iy] * wxd[ix])[..., None] * tap
    return out.astype(x.dtype)


def _resize_half(x):
    return (0.25 * (x[:, 0::2, 0::2] + x[:, 1::2, 0::2]
                    + x[:, 0::2, 1::2] + x[:, 1::2, 1::2])).astype(x.dtype)


def _axis_up2(x, axis):
    first = jax.lax.slice_in_dim(x, 0, 1, axis=axis)
    last = jax.lax.slice_in_dim(x, x.shape[axis] - 1, x.shape[axis], axis=axis)
    prev = jnp.concatenate([first, jax.lax.slice_in_dim(x, 0, x.shape[axis] - 1, axis=axis)], axis=axis)
    nxt = jnp.concatenate([jax.lax.slice_in_dim(x, 1, x.shape[axis], axis=axis), last], axis=axis)
    even = 0.25 * prev + 0.75 * x
    odd = 0.75 * x + 0.25 * nxt
    stacked = jnp.stack([even, odd], axis=axis + 1)
    shp = list(x.shape)
    shp[axis] *= 2
    return stacked.reshape(shp)


def _resize_up2(x):
    return _axis_up2(_axis_up2(x, 1), 2).astype(x.dtype)


def kernel(imgs, ctx_conv0_c1_w, ctx_conv0_c1_b, ctx_conv0_c1_a, ctx_conv0_c2_w, ctx_conv0_c2_b, ctx_conv0_c2_a, ctx_conv1_c1_w, ctx_conv1_c1_b, ctx_conv1_c1_a, ctx_conv1_c2_w, ctx_conv1_c2_b, ctx_conv1_c2_a, ctx_conv2_c1_w, ctx_conv2_c1_b, ctx_conv2_c1_a, ctx_conv2_c2_w, ctx_conv2_c2_b, ctx_conv2_c2_a, ctx_conv3_c1_w, ctx_conv3_c1_b, ctx_conv3_c1_a, ctx_conv3_c2_w, ctx_conv3_c2_b, ctx_conv3_c2_a, ctx_conv4_c1_w, ctx_conv4_c1_b, ctx_conv4_c1_a, ctx_conv4_c2_w, ctx_conv4_c2_b, ctx_conv4_c2_a, fus_conv0_c1_w, fus_conv0_c1_b, fus_conv0_c1_a, fus_conv0_c2_w, fus_conv0_c2_b, fus_conv0_c2_a, fus_down0_c1_w, fus_down0_c1_b, fus_down0_c1_a, fus_down0_c2_w, fus_down0_c2_b, fus_down0_c2_a, fus_down1_c1_w, fus_down1_c1_b, fus_down1_c1_a, fus_down1_c2_w, fus_down1_c2_b, fus_down1_c2_a, fus_down2_c1_w, fus_down2_c1_b, fus_down2_c1_a, fus_down2_c2_w, fus_down2_c2_b, fus_down2_c2_a, fus_down3_c1_w, fus_down3_c1_b, fus_down3_c1_a, fus_down3_c2_w, fus_down3_c2_b, fus_down3_c2_a, fus_up0_w, fus_up0_b, fus_up0_a, fus_up1_w, fus_up1_b, fus_up1_a, fus_up2_w, fus_up2_b, fus_up2_a, fus_up3_w, fus_up3_b, fus_up3_a, fus_conv_w, fus_conv_b, fus_conv_a):
    n, _, h, w = imgs.shape

    pooled = imgs.reshape(n, 6, h // 2, 2, w // 2, 2).mean(axis=(3, 5))
    f = jnp.stack([pooled[:, 0] - pooled[:, 3], pooled[:, 1] - pooled[:, 4],
                   pooled[:, 3] - pooled[:, 0], pooled[:, 4] - pooled[:, 1]], axis=1)
    flow = jnp.transpose(jnp.tanh(f) * 2.0, (0, 2, 3, 1))

    img0 = jnp.transpose(imgs[:, :3], (0, 2, 3, 1)).astype(jnp.bfloat16)
    img1 = jnp.transpose(imgs[:, 3:], (0, 2, 3, 1)).astype(jnp.bfloat16)

    ctx = [
        (ctx_conv0_c1_w, ctx_conv0_c1_b, ctx_conv0_c1_a,
         ctx_conv0_c2_w, ctx_conv0_c2_b, ctx_conv0_c2_a),
        (ctx_conv1_c1_w, ctx_conv1_c1_b, ctx_conv1_c1_a,
         ctx_conv1_c2_w, ctx_conv1_c2_b, ctx_conv1_c2_a),
        (ctx_conv2_c1_w, ctx_conv2_c1_b, ctx_conv2_c1_a,
         ctx_conv2_c2_w, ctx_conv2_c2_b, ctx_conv2_c2_a),
        (ctx_conv3_c1_w, ctx_conv3_c1_b, ctx_conv3_c1_a,
         ctx_conv3_c2_w, ctx_conv3_c2_b, ctx_conv3_c2_a),
        (ctx_conv4_c1_w, ctx_conv4_c1_b, ctx_conv4_c1_a,
         ctx_conv4_c2_w, ctx_conv4_c2_b, ctx_conv4_c2_a),
    ]
    xb = jnp.concatenate([img0, img1], axis=0)
    fb = jnp.concatenate([flow[..., :2], flow[..., 2:4]], axis=0)
    xb = _block(xb, ctx[0])
    xb = _block(xb, ctx[1])
    feats = []
    for lvl in range(2, 5):
        fb = _resize_half(fb) * 0.5
        feats.append(_warp(xb, fb, 1))
        xb = _block(xb, ctx[lvl])
    fb = _resize_half(fb) * 0.5
    feats.append(_warp(xb, fb, 1))
    c0 = [fz[:n] for fz in feats]
    c1 = [fz[n:] for fz in feats]

    flow_up = _resize_up2(flow) * 2.0
    w0 = _warp(img0, flow_up[..., :2], 4)
    w1 = _warp(img1, flow_up[..., 2:4], 4)
    x = jnp.concatenate([w0, w1, flow_up.astype(jnp.bfloat16)], axis=-1)
    x = _block(x, (fus_conv0_c1_w, fus_conv0_c1_b, fus_conv0_c1_a,
                   fus_conv0_c2_w, fus_conv0_c2_b, fus_conv0_c2_a))
    s0 = _block(x, (fus_down0_c1_w, fus_down0_c1_b, fus_down0_c1_a,
                    fus_down0_c2_w, fus_down0_c2_b, fus_down0_c2_a))
    s1 = _block(jnp.concatenate([s0, c0[0], c1[0]], -1),
                (fus_down1_c1_w, fus_down1_c1_b, fus_down1_c1_a,
                 fus_down1_c2_w, fus_down1_c2_b, fus_down1_c2_a))
    s2 = _block(jnp.concatenate([s1, c0[1], c1[1]], -1),
                (fus_down2_c1_w, fus_down2_c1_b, fus_down2_c1_a,
                 fus_down2_c2_w, fus_down2_c2_b, fus_down2_c2_a))
    s3 = _block(jnp.concatenate([s2, c0[2], c1[2]], -1),
                (fus_down3_c1_w, fus_down3_c1_b, fus_down3_c1_a,
                 fus_down3_c2_w, fus_down3_c2_b, fus_down3_c2_a))
    x = _deconv(jnp.concatenate([s3, c0[3], c1[3]], -1), fus_up0_w, fus_up0_b, fus_up0_a)
    x = _deconv(jnp.concatenate([x, s2], -1), fus_up1_w, fus_up1_b, fus_up1_a)
    x = _deconv(jnp.concatenate([x, s1], -1), fus_up2_w, fus_up2_b, fus_up2_a)
    x = _deconv(jnp.concatenate([x, s0], -1), fus_up3_w, fus_up3_b, fus_up3_a)
    return _final_deconv_blend(x, fus_conv_w, fus_conv_b, w0, w1)


# speedup vs baseline: 11.6689x; 3.3006x over previous
import jax
import jax.numpy as jnp
from jax.experimental import pallas as pl
from jax.experimental.pallas import tpu as pltpu


def _w_s2d(w):
    cout, cin = w.shape[0], w.shape[1]
    big = jnp.zeros((2, 2, 2, 2, cin, cout), w.dtype)
    for i in range(3):
        for j in range(3):
            big = big.at[i // 2, j // 2, i % 2, j % 2].set(jnp.transpose(w[:, :, i, j]))
    return big.reshape(4, 4 * cin, cout)


def _w_conv1(w):
    return jnp.transpose(w, (2, 3, 1, 0)).reshape(9, w.shape[1], w.shape[0])


def _w_deconv(w):
    cin, cout = w.shape[0], w.shape[1]
    big = jnp.zeros((3, 3, cin, 2, 2, cout), w.dtype)
    for py in range(2):
        for r in range(2):
            for px in range(2):
                for s in range(2):
                    big = big.at[py + r, px + s, :, py, px, :].set(
                        w[:, :, 3 - py - 2 * r, 3 - px - 2 * s])
    return big.reshape(9, cin, 4 * cout)


def _ring(x):
    return jnp.pad(x, ((0, 0), (1, 1), (1, 1), (0, 0)))


def _valid(fr):
    return fr[:, 1:-1, 1:-1, :]


def _s2d_pad(fr):
    n, H, _, c = fr.shape
    q = H // 2
    v = fr.reshape(n, q, 2, q, 2, c).transpose(0, 1, 3, 2, 4, 5).reshape(n, q, q, 4 * c)
    return jnp.pad(v, ((0, 0), (0, 1), (0, 1), (0, 0)))


def _d2s_valid(fr, cout):
    n, P = fr.shape[0], fr.shape[1]
    h = P - 2
    v = _valid(fr).reshape(n, h, h, 2, 2, cout)
    return v.transpose(0, 1, 3, 2, 4, 5).reshape(n, 2 * h, 2 * h, cout)


_CP = pltpu.CompilerParams(dimension_semantics=("parallel",),
                           vmem_limit_bytes=60 * 1024 * 1024)


def _interior_mask(P):
    PP = P * P
    r = jax.lax.broadcasted_iota(jnp.int32, (PP, 1), 0)
    row = r // P
    col = r % P
    return (row >= 1) & (row <= P - 2) & (col >= 1) & (col <= P - 2)


def _tap_accum(acc_ref, src_slice, wt_ref, taps, P, PP):
    for t, (dy, dx) in enumerate(taps):
        k = (dy - 1) * P + (dx - 1)
        lo = max(0, -k)
        hi = PP - max(0, k)
        acc_ref[lo:hi, :] += jnp.dot(src_slice(lo + k, hi + k), wt_ref[t],
                                     preferred_element_type=jnp.float32)


_T22 = [(u, v) for u in range(2) for v in range(2)]
_T33 = [(d, e) for d in range(3) for e in range(3)]


def _mk_block_body(ns, P):
    PP = P * P

    def body(*refs):
        srcs = refs[:ns]
        w1s = refs[ns:2 * ns]
        b1, a1, w2, b2, a2, out = refs[2 * ns:2 * ns + 6]
        acc, y1 = refs[2 * ns + 6:]
        inside = _interior_mask(P)
        acc[...] = jnp.zeros_like(acc)
        for s in range(ns):
            _tap_accum(acc, (lambda a, b, _r=srcs[s]: _r[0, a:b, :]), w1s[s], _T22, P, PP)
        y = acc[...] + b1[...]
        y = jnp.where(y >= 0.0, y, y * a1[...])
        y1[...] = jnp.where(inside, y, 0.0).astype(y1.dtype)
        acc[...] = jnp.zeros_like(acc)
        _tap_accum(acc, (lambda a, b: y1[a:b, :]), w2, _T33, P, PP)
        z = acc[...] + b2[...]
        z = jnp.where(z >= 0.0, z, z * a2[...])
        out[0] = jnp.where(inside, z, 0.0).astype(out.dtype)

    return body


def _block_f(srcs, w1, b1, a1, w2, b2, a2):
    n, H = srcs[0].shape[0], srcs[0].shape[1]
    P = H // 2 + 1
    PP = P * P
    cout = w1.shape[0]
    cs = [s.shape[-1] for s in srcs]
    flat = [_s2d_pad(s).reshape(n, PP, 4 * c).astype(jnp.bfloat16)
            for s, c in zip(srcs, cs)]
    offs = [sum(cs[:i]) for i in range(len(cs))]
    w1s = [_w_s2d(w1[:, o:o + c]).astype(jnp.bfloat16) for o, c in zip(offs, cs)]
    w2t = _w_conv1(w2).astype(jnp.bfloat16)
    b1r = b1.astype(jnp.float32).reshape(1, cout)
    a1r = a1.astype(jnp.float32).reshape(1, cout)
    b2r = b2.astype(jnp.float32).reshape(1, cout)
    a2r = a2.astype(jnp.float32).reshape(1, cout)
    ns = len(srcs)
    out = pl.pallas_call(
        _mk_block_body(ns, P),
        out_shape=jax.ShapeDtypeStruct((n, PP, cout), jnp.bfloat16),
        grid=(n,),
        in_specs=(
            [pl.BlockSpec((1, PP, 4 * c), lambda i: (i, 0, 0)) for c in cs]
            + [pl.BlockSpec((4, 4 * c, cout), lambda i: (0, 0, 0)) for c in cs]
            + [pl.BlockSpec((1, cout), lambda i: (0, 0)),
               pl.BlockSpec((1, cout), lambda i: (0, 0)),
               pl.BlockSpec((9, cout, cout), lambda i: (0, 0, 0)),
               pl.BlockSpec((1, cout), lambda i: (0, 0)),
               pl.BlockSpec((1, cout), lambda i: (0, 0))]
        ),
        out_specs=pl.BlockSpec((1, PP, cout), lambda i: (i, 0, 0)),
        scratch_shapes=[pltpu.VMEM((PP, cout), jnp.float32),
                        pltpu.VMEM((PP, cout), jnp.bfloat16)],
        compiler_params=_CP,
    )(*flat, *w1s, b1r, a1r, w2t, b2r, a2r)
    return out.reshape(n, P, P, cout)


def _mk_deconv_body(ns, P, prelu):
    PP = P * P

    def body(*refs):
        srcs = refs[:ns]
        ws = refs[ns:2 * ns]
        b, a, out = refs[2 * ns:2 * ns + 3]
        acc = refs[2 * ns + 3]
        acc[...] = jnp.zeros_like(acc)
        for s in range(ns):
            _tap_accum(acc, (lambda a, b, _r=srcs[s]: _r[0, a:b, :]), ws[s], _T33, P, PP)
        z = acc[...] + b[...]
        if prelu:
            z = jnp.where(z >= 0.0, z, z * a[...])
        out[0] = z.astype(out.dtype)

    return body


def _deconv_f(srcs, w, b, a, prelu=True):
    n, P = srcs[0].shape[0], srcs[0].shape[1]
    PP = P * P
    cout = w.shape[1]
    cs = [s.shape[-1] for s in srcs]
    flat = [s.reshape(n, PP, c).astype(jnp.bfloat16) for s, c in zip(srcs, cs)]
    offs = [sum(cs[:i]) for i in range(len(cs))]
    ws = [_w_deconv(w[o:o + c]).astype(jnp.bfloat16) for o, c in zip(offs, cs)]
    br = jnp.tile(b, 4).astype(jnp.float32).reshape(1, 4 * cout)
    ar = jnp.tile(a, 4).astype(jnp.float32).reshape(1, 4 * cout)
    ns = len(srcs)
    out = pl.pallas_call(
        _mk_deconv_body(ns, P, prelu),
        out_shape=jax.ShapeDtypeStruct((n, PP, 4 * cout), jnp.bfloat16),
        grid=(n,),
        in_specs=(
            [pl.BlockSpec((1, PP, c), lambda i: (i, 0, 0)) for c in cs]
            + [pl.BlockSpec((9, c, 4 * cout), lambda i: (0, 0, 0)) for c in cs]
            + [pl.BlockSpec((1, 4 * cout), lambda i: (0, 0)),
               pl.BlockSpec((1, 4 * cout), lambda i: (0, 0))]
        ),
        out_specs=pl.BlockSpec((1, PP, 4 * cout), lambda i: (i, 0, 0)),
        scratch_shapes=[pltpu.VMEM((PP, 4 * cout), jnp.float32)],
        compiler_params=_CP,
    )(*flat, *ws, br, ar)
    return out.reshape(n, P, P, 4 * cout)


def _mk_final_body(P):
    PP = P * P

    def body(src, wt, bb, bmat, w0f, w1f, out, acc):
        acc[...] = jnp.zeros_like(acc)
        _tap_accum(acc, (lambda a, b: src[0, a:b, :]), wt, _T33, P, PP)
        refine = (acc[...] + bb[...]).astype(jnp.bfloat16).astype(jnp.float32)
        s = jax.nn.sigmoid(refine)
        mask = jnp.dot(s, bmat[...], preferred_element_type=jnp.float32)
        w0 = w0f[0].astype(jnp.float32)
        w1 = w1f[0].astype(jnp.float32)
        merged = w0 * mask + w1 * (1.0 - mask) + (s * 2.0 - 1.0)
        out[0] = jnp.clip(merged, 0.0, 1.0)

    return body


def _phase_frame(img):
    n, hh = img.shape[0], img.shape[1]
    h = hh // 2
    v = img.reshape(n, h, 2, h, 2, 3).transpose(0, 1, 3, 2, 4, 5)
    v = jnp.pad(v, ((0, 0),) * 5 + ((0, 1),)).reshape(n, h, h, 16)
    return _ring(v).reshape(n, (h + 2) * (h + 2), 16).astype(jnp.bfloat16)


def _final_f(src, w, b, w0, w1):
    n, P = src.shape[0], src.shape[1]
    PP = P * P
    h = P - 2
    flat = src.reshape(n, PP, src.shape[-1]).astype(jnp.bfloat16)
    wt = _w_deconv(w).astype(jnp.bfloat16)
    bb = jnp.tile(b, 4).astype(jnp.float32).reshape(1, 16)
    bmat = jnp.zeros((16, 16), jnp.float32)
    for p in range(4):
        bmat = bmat.at[4 * p + 3, 4 * p:4 * p + 3].set(1.0)
    out = pl.pallas_call(
        _mk_final_body(P),
        out_shape=jax.ShapeDtypeStruct((n, PP, 16), jnp.float32),
        grid=(n,),
        in_specs=[
            pl.BlockSpec((1, PP, 32), lambda i: (i, 0, 0)),
            pl.BlockSpec((9, 32, 16), lambda i: (0, 0, 0)),
            pl.BlockSpec((1, 16), lambda i: (0, 0)),
            pl.BlockSpec((16, 16), lambda i: (0, 0)),
            pl.BlockSpec((1, PP, 16), lambda i: (i, 0, 0)),
            pl.BlockSpec((1, PP, 16), lambda i: (i, 0, 0)),
        ],
        out_specs=pl.BlockSpec((1, PP, 16), lambda i: (i, 0, 0)),
        scratch_shapes=[pltpu.VMEM((PP, 16), jnp.float32)],
        compiler_params=_CP,
    )(flat, wt, bb, bmat, _phase_frame(w0), _phase_frame(w1))
    v = out.reshape(n, P, P, 16)[:, 1:-1, 1:-1, :].reshape(n, h, h, 2, 2, 4)[..., :3]
    v = v.transpose(0, 1, 3, 2, 4, 5).reshape(n, 2 * h, 2 * h, 3)
    return jnp.transpose(v, (0, 3, 1, 2))


def _warp(x, flow, radius):
    n, h, w, c = x.shape
    r = radius
    gy = jnp.arange(h, dtype=jnp.float32)[None, :, None]
    gx = jnp.arange(w, dtype=jnp.float32)[None, None, :]
    sx = jnp.clip(gx + flow[..., 0], 0.0, w - 1.0)
    sy = jnp.clip(gy + flow[..., 1], 0.0, h - 1.0)
    x0 = jnp.floor(sx)
    y0 = jnp.floor(sy)
    wx = sx - x0
    wy = sy - y0
    x0i = x0.astype(jnp.int32)
    y0i = y0.astype(jnp.int32)
    x1i = jnp.minimum(x0i + 1, w - 1)
    y1i = jnp.minimum(y0i + 1, h - 1)
    dx0 = x0i - gx.astype(jnp.int32)
    dx1 = x1i - gx.astype(jnp.int32)
    dy0 = y0i - gy.astype(jnp.int32)
    dy1 = y1i - gy.astype(jnp.int32)
    wxd = [jnp.where(dx0 == d, 1.0 - wx, 0.0) + jnp.where(dx1 == d, wx, 0.0)
           for d in range(-r, r + 1)]
    wyd = [jnp.where(dy0 == d, 1.0 - wy, 0.0) + jnp.where(dy1 == d, wy, 0.0)
           for d in range(-r, r + 1)]
    xp = jnp.pad(x, ((0, 0), (r, r), (r, r), (0, 0)))
    out = jnp.zeros((n, h, w, c), jnp.float32)
    for iy, dy in enumerate(range(-r, r + 1)):
        for ix, dx in enumerate(range(-r, r + 1)):
            tap = xp[:, r + dy:r + dy + h, r + dx:r + dx + w, :]
            out = out + (wyd[iy] * wxd[ix])[..., None] * tap
    return out.astype(x.dtype)


def _resize_half(x):
    return (0.25 * (x[:, 0::2, 0::2] + x[:, 1::2, 0::2]
                    + x[:, 0::2, 1::2] + x[:, 1::2, 1::2])).astype(x.dtype)


def _axis_up2(x, axis):
    first = jax.lax.slice_in_dim(x, 0, 1, axis=axis)
    last = jax.lax.slice_in_dim(x, x.shape[axis] - 1, x.shape[axis], axis=axis)
    prev = jnp.concatenate([first, jax.lax.slice_in_dim(x, 0, x.shape[axis] - 1, axis=axis)], axis=axis)
    nxt = jnp.concatenate([jax.lax.slice_in_dim(x, 1, x.shape[axis], axis=axis), last], axis=axis)
    even = 0.25 * prev + 0.75 * x
    odd = 0.75 * x + 0.25 * nxt
    stacked = jnp.stack([even, odd], axis=axis + 1)
    shp = list(x.shape)
    shp[axis] *= 2
    return stacked.reshape(shp)


def _resize_up2(x):
    return _axis_up2(_axis_up2(x, 1), 2).astype(x.dtype)


def kernel(imgs, ctx_conv0_c1_w, ctx_conv0_c1_b, ctx_conv0_c1_a, ctx_conv0_c2_w, ctx_conv0_c2_b, ctx_conv0_c2_a, ctx_conv1_c1_w, ctx_conv1_c1_b, ctx_conv1_c1_a, ctx_conv1_c2_w, ctx_conv1_c2_b, ctx_conv1_c2_a, ctx_conv2_c1_w, ctx_conv2_c1_b, ctx_conv2_c1_a, ctx_conv2_c2_w, ctx_conv2_c2_b, ctx_conv2_c2_a, ctx_conv3_c1_w, ctx_conv3_c1_b, ctx_conv3_c1_a, ctx_conv3_c2_w, ctx_conv3_c2_b, ctx_conv3_c2_a, ctx_conv4_c1_w, ctx_conv4_c1_b, ctx_conv4_c1_a, ctx_conv4_c2_w, ctx_conv4_c2_b, ctx_conv4_c2_a, fus_conv0_c1_w, fus_conv0_c1_b, fus_conv0_c1_a, fus_conv0_c2_w, fus_conv0_c2_b, fus_conv0_c2_a, fus_down0_c1_w, fus_down0_c1_b, fus_down0_c1_a, fus_down0_c2_w, fus_down0_c2_b, fus_down0_c2_a, fus_down1_c1_w, fus_down1_c1_b, fus_down1_c1_a, fus_down1_c2_w, fus_down1_c2_b, fus_down1_c2_a, fus_down2_c1_w, fus_down2_c1_b, fus_down2_c1_a, fus_down2_c2_w, fus_down2_c2_b, fus_down2_c2_a, fus_down3_c1_w, fus_down3_c1_b, fus_down3_c1_a, fus_down3_c2_w, fus_down3_c2_b, fus_down3_c2_a, fus_up0_w, fus_up0_b, fus_up0_a, fus_up1_w, fus_up1_b, fus_up1_a, fus_up2_w, fus_up2_b, fus_up2_a, fus_up3_w, fus_up3_b, fus_up3_a, fus_conv_w, fus_conv_b, fus_conv_a):
    n, _, h, w = imgs.shape

    pooled = imgs.reshape(n, 6, h // 2, 2, w // 2, 2).mean(axis=(3, 5))
    f = jnp.stack([pooled[:, 0] - pooled[:, 3], pooled[:, 1] - pooled[:, 4],
                   pooled[:, 3] - pooled[:, 0], pooled[:, 4] - pooled[:, 1]], axis=1)
    flow = jnp.transpose(jnp.tanh(f) * 2.0, (0, 2, 3, 1))

    img0 = jnp.transpose(imgs[:, :3], (0, 2, 3, 1)).astype(jnp.bfloat16)
    img1 = jnp.transpose(imgs[:, 3:], (0, 2, 3, 1)).astype(jnp.bfloat16)

    ctx = [
        (ctx_conv0_c1_w, ctx_conv0_c1_b, ctx_conv0_c1_a,
         ctx_conv0_c2_w, ctx_conv0_c2_b, ctx_conv0_c2_a),
        (ctx_conv1_c1_w, ctx_conv1_c1_b, ctx_conv1_c1_a,
         ctx_conv1_c2_w, ctx_conv1_c2_b, ctx_conv1_c2_a),
        (ctx_conv2_c1_w, ctx_conv2_c1_b, ctx_conv2_c1_a,
         ctx_conv2_c2_w, ctx_conv2_c2_b, ctx_conv2_c2_a),
        (ctx_conv3_c1_w, ctx_conv3_c1_b, ctx_conv3_c1_a,
         ctx_conv3_c2_w, ctx_conv3_c2_b, ctx_conv3_c2_a),
        (ctx_conv4_c1_w, ctx_conv4_c1_b, ctx_conv4_c1_a,
         ctx_conv4_c2_w, ctx_conv4_c2_b, ctx_conv4_c2_a),
    ]
    xb = _ring(jnp.concatenate([img0, img1], axis=0))
    fb = jnp.concatenate([flow[..., :2], flow[..., 2:4]], axis=0)
    xb = _block_f([xb], *ctx[0])
    xb = _block_f([xb], *ctx[1])
    feats = []
    for lvl in range(2, 5):
        fb = _resize_half(fb) * 0.5
        feats.append(_warp(_valid(xb), fb, 1))
        xb = _block_f([xb], *ctx[lvl])
    fb = _resize_half(fb) * 0.5
    feats.append(_warp(_valid(xb), fb, 1))
    c0 = [fz[:n] for fz in feats]
    c1 = [fz[n:] for fz in feats]

    flow_up = _resize_up2(flow) * 2.0
    w0 = _warp(img0, flow_up[..., :2], 4)
    w1 = _warp(img1, flow_up[..., 2:4], 4)
    x = _block_f([_ring(w0), _ring(w1), _ring(flow_up.astype(jnp.bfloat16))],
                 fus_conv0_c1_w, fus_conv0_c1_b, fus_conv0_c1_a,
                 fus_conv0_c2_w, fus_conv0_c2_b, fus_conv0_c2_a)
    s0 = _block_f([x], fus_down0_c1_w, fus_down0_c1_b, fus_down0_c1_a,
                  fus_down0_c2_w, fus_down0_c2_b, fus_down0_c2_a)
    s1 = _block_f([s0, _ring(c0[0]), _ring(c1[0])],
                  fus_down1_c1_w, fus_down1_c1_b, fus_down1_c1_a,
                  fus_down1_c2_w, fus_down1_c2_b, fus_down1_c2_a)
    s2 = _block_f([s1, _ring(c0[1]), _ring(c1[1])],
                  fus_down2_c1_w, fus_down2_c1_b, fus_down2_c1_a,
                  fus_down2_c2_w, fus_down2_c2_b, fus_down2_c2_a)
    s3 = _block_f([s2, _ring(c0[2]), _ring(c1[2])],
                  fus_down3_c1_w, fus_down3_c1_b, fus_down3_c1_a,
                  fus_down3_c2_w, fus_down3_c2_b, fus_down3_c2_a)
    u = _deconv_f([s3, _ring(c0[3]), _ring(c1[3])], fus_up0_w, fus_up0_b, fus_up0_a)
    u = _deconv_f([_ring(_d2s_valid(u, 256)), s2], fus_up1_w, fus_up1_b, fus_up1_a)
    u = _deconv_f([_ring(_d2s_valid(u, 128)), s1], fus_up2_w, fus_up2_b, fus_up2_a)
    u = _deconv_f([_ring(_d2s_valid(u, 64)), s0], fus_up3_w, fus_up3_b, fus_up3_a)
    x = _ring(_d2s_valid(u, 32))
    return _final_f(x, fus_conv_w, fus_conv_b, w0, w1)


# speedup vs baseline: 14.8391x; 1.2717x over previous
import jax
import jax.numpy as jnp
from jax.experimental import pallas as pl
from jax.experimental.pallas import tpu as pltpu


def _w_s2d(w):
    cout, cin = w.shape[0], w.shape[1]
    big = jnp.zeros((2, 2, 2, 2, cin, cout), w.dtype)
    for i in range(3):
        for j in range(3):
            big = big.at[i // 2, j // 2, i % 2, j % 2].set(jnp.transpose(w[:, :, i, j]))
    return big.reshape(4, 4 * cin, cout)


def _w_conv1(w):
    return jnp.transpose(w, (2, 3, 1, 0)).reshape(9, w.shape[1], w.shape[0])


def _w_deconv(w):
    cin, cout = w.shape[0], w.shape[1]
    big = jnp.zeros((3, 3, cin, 2, 2, cout), w.dtype)
    for py in range(2):
        for r in range(2):
            for px in range(2):
                for s in range(2):
                    big = big.at[py + r, px + s, :, py, px, :].set(
                        w[:, :, 3 - py - 2 * r, 3 - px - 2 * s])
    return big.reshape(9, cin, 4 * cout)


def _ring(x):
    return jnp.pad(x, ((0, 0), (1, 1), (1, 1), (0, 0)))


def _valid(fr):
    return fr[:, 1:-1, 1:-1, :]


def _s2d_pad(fr):
    n, H, _, c = fr.shape
    q = H // 2
    v = fr.reshape(n, q, 2, q, 2, c).transpose(0, 1, 3, 2, 4, 5).reshape(n, q, q, 4 * c)
    return jnp.pad(v, ((0, 0), (0, 1), (0, 1), (0, 0)))


def _d2s_valid(fr, cout):
    n, P = fr.shape[0], fr.shape[1]
    h = P - 2
    v = _valid(fr).reshape(n, h, h, 2, 2, cout)
    return v.transpose(0, 1, 3, 2, 4, 5).reshape(n, 2 * h, 2 * h, cout)


_CP = pltpu.CompilerParams(dimension_semantics=("parallel",),
                           vmem_limit_bytes=60 * 1024 * 1024)


def _interior_mask(P):
    PP = P * P
    r = jax.lax.broadcasted_iota(jnp.int32, (PP, 1), 0)
    row = r // P
    col = r % P
    return (row >= 1) & (row <= P - 2) & (col >= 1) & (col <= P - 2)


def _tap_accum(acc_ref, src_slice, wt_ref, taps, P, PP):
    for t, (dy, dx) in enumerate(taps):
        k = (dy - 1) * P + (dx - 1)
        lo = max(0, -k)
        hi = PP - max(0, k)
        acc_ref[lo:hi, :] += jnp.dot(src_slice(lo + k, hi + k), wt_ref[t],
                                     preferred_element_type=jnp.float32)


_T22 = [(u, v) for u in range(2) for v in range(2)]
_T33 = [(d, e) for d in range(3) for e in range(3)]


def _mk_block_body(ns, P):
    PP = P * P

    def body(*refs):
        srcs = refs[:ns]
        w1s = refs[ns:2 * ns]
        b1, a1, w2, b2, a2, out = refs[2 * ns:2 * ns + 6]
        acc, y1 = refs[2 * ns + 6:]
        inside = _interior_mask(P)
        acc[...] = jnp.zeros_like(acc)
        for s in range(ns):
            _tap_accum(acc, (lambda a, b, _r=srcs[s]: _r[0, a:b, :]), w1s[s], _T22, P, PP)
        y = acc[...] + b1[...]
        y = jnp.where(y >= 0.0, y, y * a1[...])
        y1[...] = jnp.where(inside, y, 0.0).astype(y1.dtype)
        acc[...] = jnp.zeros_like(acc)
        _tap_accum(acc, (lambda a, b: y1[a:b, :]), w2, _T33, P, PP)
        z = acc[...] + b2[...]
        z = jnp.where(z >= 0.0, z, z * a2[...])
        out[0] = jnp.where(inside, z, 0.0).astype(out.dtype)

    return body


def _block_f(srcs, w1, b1, a1, w2, b2, a2):
    n, H = srcs[0].shape[0], srcs[0].shape[1]
    P = H // 2 + 1
    PP = P * P
    cout = w1.shape[0]
    cs = [s.shape[-1] for s in srcs]
    flat = [_s2d_pad(s).reshape(n, PP, 4 * c).astype(jnp.bfloat16)
            for s, c in zip(srcs, cs)]
    offs = [sum(cs[:i]) for i in range(len(cs))]
    w1s = [_w_s2d(w1[:, o:o + c]).astype(jnp.bfloat16) for o, c in zip(offs, cs)]
    w2t = _w_conv1(w2).astype(jnp.bfloat16)
    b1r = b1.astype(jnp.float32).reshape(1, cout)
    a1r = a1.astype(jnp.float32).reshape(1, cout)
    b2r = b2.astype(jnp.float32).reshape(1, cout)
    a2r = a2.astype(jnp.float32).reshape(1, cout)
    ns = len(srcs)
    out = pl.pallas_call(
        _mk_block_body(ns, P),
        out_shape=jax.ShapeDtypeStruct((n, PP, cout), jnp.bfloat16),
        grid=(n,),
        in_specs=(
            [pl.BlockSpec((1, PP, 4 * c), lambda i: (i, 0, 0)) for c in cs]
            + [pl.BlockSpec((4, 4 * c, cout), lambda i: (0, 0, 0)) for c in cs]
            + [pl.BlockSpec((1, cout), lambda i: (0, 0)),
               pl.BlockSpec((1, cout), lambda i: (0, 0)),
               pl.BlockSpec((9, cout, cout), lambda i: (0, 0, 0)),
               pl.BlockSpec((1, cout), lambda i: (0, 0)),
               pl.BlockSpec((1, cout), lambda i: (0, 0))]
        ),
        out_specs=pl.BlockSpec((1, PP, cout), lambda i: (i, 0, 0)),
        scratch_shapes=[pltpu.VMEM((PP, cout), jnp.float32),
                        pltpu.VMEM((PP, cout), jnp.bfloat16)],
        compiler_params=_CP,
    )(*flat, *w1s, b1r, a1r, w2t, b2r, a2r)
    return out.reshape(n, P, P, cout)


def _mk_deconv_body(ns, P, prelu):
    PP = P * P

    def body(*refs):
        srcs = refs[:ns]
        ws = refs[ns:2 * ns]
        b, a, out = refs[2 * ns:2 * ns + 3]
        acc = refs[2 * ns + 3]
        acc[...] = jnp.zeros_like(acc)
        for s in range(ns):
            _tap_accum(acc, (lambda a, b, _r=srcs[s]: _r[0, a:b, :]), ws[s], _T33, P, PP)
        z = acc[...] + b[...]
        if prelu:
            z = jnp.where(z >= 0.0, z, z * a[...])
        out[0] = z.astype(out.dtype)

    return body


def _deconv_f(srcs, w, b, a, prelu=True):
    n, P = srcs[0].shape[0], srcs[0].shape[1]
    PP = P * P
    cout = w.shape[1]
    cs = [s.shape[-1] for s in srcs]
    flat = [s.reshape(n, PP, c).astype(jnp.bfloat16) for s, c in zip(srcs, cs)]
    offs = [sum(cs[:i]) for i in range(len(cs))]
    ws = [_w_deconv(w[o:o + c]).astype(jnp.bfloat16) for o, c in zip(offs, cs)]
    br = jnp.tile(b, 4).astype(jnp.float32).reshape(1, 4 * cout)
    ar = jnp.tile(a, 4).astype(jnp.float32).reshape(1, 4 * cout)
    ns = len(srcs)
    out = pl.pallas_call(
        _mk_deconv_body(ns, P, prelu),
        out_shape=jax.ShapeDtypeStruct((n, PP, 4 * cout), jnp.bfloat16),
        grid=(n,),
        in_specs=(
            [pl.BlockSpec((1, PP, c), lambda i: (i, 0, 0)) for c in cs]
            + [pl.BlockSpec((9, c, 4 * cout), lambda i: (0, 0, 0)) for c in cs]
            + [pl.BlockSpec((1, 4 * cout), lambda i: (0, 0)),
               pl.BlockSpec((1, 4 * cout), lambda i: (0, 0))]
        ),
        out_specs=pl.BlockSpec((1, PP, 4 * cout), lambda i: (i, 0, 0)),
        scratch_shapes=[pltpu.VMEM((PP, 4 * cout), jnp.float32)],
        compiler_params=_CP,
    )(*flat, *ws, br, ar)
    return out.reshape(n, P, P, 4 * cout)


def _mk_final_body(P):
    PP = P * P

    def body(src, wt, bb, bmat, w0f, w1f, out, acc):
        acc[...] = jnp.zeros_like(acc)
        _tap_accum(acc, (lambda a, b: src[0, a:b, :]), wt, _T33, P, PP)
        refine = (acc[...] + bb[...]).astype(jnp.bfloat16).astype(jnp.float32)
        s = jax.nn.sigmoid(refine)
        mask = jnp.dot(s, bmat[...], preferred_element_type=jnp.float32)
        w0 = w0f[0].astype(jnp.float32)
        w1 = w1f[0].astype(jnp.float32)
        merged = w0 * mask + w1 * (1.0 - mask) + (s * 2.0 - 1.0)
        out[0] = jnp.clip(merged, 0.0, 1.0)

    return body


def _phase_frame(img):
    n, hh = img.shape[0], img.shape[1]
    h = hh // 2
    v = img.reshape(n, h, 2, h, 2, 3).transpose(0, 1, 3, 2, 4, 5)
    v = jnp.pad(v, ((0, 0),) * 5 + ((0, 1),)).reshape(n, h, h, 16)
    return _ring(v).reshape(n, (h + 2) * (h + 2), 16).astype(jnp.bfloat16)


def _final_f(src, w, b, w0, w1):
    n, P = src.shape[0], src.shape[1]
    PP = P * P
    h = P - 2
    flat = src.reshape(n, PP, src.shape[-1]).astype(jnp.bfloat16)
    wt = _w_deconv(w).astype(jnp.bfloat16)
    bb = jnp.tile(b, 4).astype(jnp.float32).reshape(1, 16)
    bmat = jnp.zeros((16, 16), jnp.float32)
    for p in range(4):
        bmat = bmat.at[4 * p + 3, 4 * p:4 * p + 3].set(1.0)
    out = pl.pallas_call(
        _mk_final_body(P),
        out_shape=jax.ShapeDtypeStruct((n, PP, 16), jnp.float32),
        grid=(n,),
        in_specs=[
            pl.BlockSpec((1, PP, 32), lambda i: (i, 0, 0)),
            pl.BlockSpec((9, 32, 16), lambda i: (0, 0, 0)),
            pl.BlockSpec((1, 16), lambda i: (0, 0)),
            pl.BlockSpec((16, 16), lambda i: (0, 0)),
            pl.BlockSpec((1, PP, 16), lambda i: (i, 0, 0)),
            pl.BlockSpec((1, PP, 16), lambda i: (i, 0, 0)),
        ],
        out_specs=pl.BlockSpec((1, PP, 16), lambda i: (i, 0, 0)),
        scratch_shapes=[pltpu.VMEM((PP, 16), jnp.float32)],
        compiler_params=_CP,
    )(flat, wt, bb, bmat, _phase_frame(w0), _phase_frame(w1))
    v = out.reshape(n, P, P, 16)[:, 1:-1, 1:-1, :].reshape(n, h, h, 2, 2, 4)[..., :3]
    v = v.transpose(0, 1, 3, 2, 4, 5).reshape(n, 2 * h, 2 * h, 3)
    return jnp.transpose(v, (0, 3, 1, 2))


def _warp(x, flow, radius):
    n, h, w, c = x.shape
    r = radius
    gy = jnp.arange(h, dtype=jnp.float32)[None, :, None]
    gx = jnp.arange(w, dtype=jnp.float32)[None, None, :]
    sx = jnp.clip(gx + flow[..., 0], 0.0, w - 1.0)
    sy = jnp.clip(gy + flow[..., 1], 0.0, h - 1.0)
    x0 = jnp.floor(sx)
    y0 = jnp.floor(sy)
    wx = sx - x0
    wy = sy - y0
    x0i = x0.astype(jnp.int32)
    y0i = y0.astype(jnp.int32)
    x1i = jnp.minimum(x0i + 1, w - 1)
    y1i = jnp.minimum(y0i + 1, h - 1)
    dx0 = x0i - gx.astype(jnp.int32)
    dx1 = x1i - gx.astype(jnp.int32)
    dy0 = y0i - gy.astype(jnp.int32)
    dy1 = y1i - gy.astype(jnp.int32)
    wxd = [jnp.where(dx0 == d, 1.0 - wx, 0.0) + jnp.where(dx1 == d, wx, 0.0)
           for d in range(-r, r + 1)]
    wyd = [jnp.where(dy0 == d, 1.0 - wy, 0.0) + jnp.where(dy1 == d, wy, 0.0)
           for d in range(-r, r + 1)]
    xp = jnp.pad(x, ((0, 0), (r, r), (r, r), (0, 0)))
    out = jnp.zeros((n, h, w, c), jnp.float32)
    for iy, dy in enumerate(range(-r, r + 1)):
        for ix, dx in enumerate(range(-r, r + 1)):
            tap = xp[:, r + dy:r + dy + h, r + dx:r + dx + w, :]
            out = out + (wyd[iy] * wxd[ix])[..., None] * tap
    return out.astype(x.dtype)


def _resize_half(x):
    return (0.25 * (x[:, 0::2, 0::2] + x[:, 1::2, 0::2]
                    + x[:, 0::2, 1::2] + x[:, 1::2, 1::2])).astype(x.dtype)


def _axis_up2(x, axis):
    first = jax.lax.slice_in_dim(x, 0, 1, axis=axis)
    last = jax.lax.slice_in_dim(x, x.shape[axis] - 1, x.shape[axis], axis=axis)
    prev = jnp.concatenate([first, jax.lax.slice_in_dim(x, 0, x.shape[axis] - 1, axis=axis)], axis=axis)
    nxt = jnp.concatenate([jax.lax.slice_in_dim(x, 1, x.shape[axis], axis=axis), last], axis=axis)
    even = 0.25 * prev + 0.75 * x
    odd = 0.75 * x + 0.25 * nxt
    stacked = jnp.stack([even, odd], axis=axis + 1)
    shp = list(x.shape)
    shp[axis] *= 2
    return stacked.reshape(shp)


def _resize_up2(x):
    return _axis_up2(_axis_up2(x, 1), 2).astype(x.dtype)


def kernel(imgs, ctx_conv0_c1_w, ctx_conv0_c1_b, ctx_conv0_c1_a, ctx_conv0_c2_w, ctx_conv0_c2_b, ctx_conv0_c2_a, ctx_conv1_c1_w, ctx_conv1_c1_b, ctx_conv1_c1_a, ctx_conv1_c2_w, ctx_conv1_c2_b, ctx_conv1_c2_a, ctx_conv2_c1_w, ctx_conv2_c1_b, ctx_conv2_c1_a, ctx_conv2_c2_w, ctx_conv2_c2_b, ctx_conv2_c2_a, ctx_conv3_c1_w, ctx_conv3_c1_b, ctx_conv3_c1_a, ctx_conv3_c2_w, ctx_conv3_c2_b, ctx_conv3_c2_a, ctx_conv4_c1_w, ctx_conv4_c1_b, ctx_conv4_c1_a, ctx_conv4_c2_w, ctx_conv4_c2_b, ctx_conv4_c2_a, fus_conv0_c1_w, fus_conv0_c1_b, fus_conv0_c1_a, fus_conv0_c2_w, fus_conv0_c2_b, fus_conv0_c2_a, fus_down0_c1_w, fus_down0_c1_b, fus_down0_c1_a, fus_down0_c2_w, fus_down0_c2_b, fus_down0_c2_a, fus_down1_c1_w, fus_down1_c1_b, fus_down1_c1_a, fus_down1_c2_w, fus_down1_c2_b, fus_down1_c2_a, fus_down2_c1_w, fus_down2_c1_b, fus_down2_c1_a, fus_down2_c2_w, fus_down2_c2_b, fus_down2_c2_a, fus_down3_c1_w, fus_down3_c1_b, fus_down3_c1_a, fus_down3_c2_w, fus_down3_c2_b, fus_down3_c2_a, fus_up0_w, fus_up0_b, fus_up0_a, fus_up1_w, fus_up1_b, fus_up1_a, fus_up2_w, fus_up2_b, fus_up2_a, fus_up3_w, fus_up3_b, fus_up3_a, fus_conv_w, fus_conv_b, fus_conv_a):
    n, _, h, w = imgs.shape

    pooled = imgs.reshape(n, 6, h // 2, 2, w // 2, 2).mean(axis=(3, 5))
    f = jnp.stack([pooled[:, 0] - pooled[:, 3], pooled[:, 1] - pooled[:, 4],
                   pooled[:, 3] - pooled[:, 0], pooled[:, 4] - pooled[:, 1]], axis=1)
    flow = jnp.transpose(jnp.tanh(f) * 2.0, (0, 2, 3, 1))

    img0 = jnp.transpose(imgs[:, :3], (0, 2, 3, 1)).astype(jnp.bfloat16)
    img1 = jnp.transpose(imgs[:, 3:], (0, 2, 3, 1)).astype(jnp.bfloat16)

    ctx = [
        (ctx_conv0_c1_w, ctx_conv0_c1_b, ctx_conv0_c1_a,
         ctx_conv0_c2_w, ctx_conv0_c2_b, ctx_conv0_c2_a),
        (ctx_conv1_c1_w, ctx_conv1_c1_b, ctx_conv1_c1_a,
         ctx_conv1_c2_w, ctx_conv1_c2_b, ctx_conv1_c2_a),
        (ctx_conv2_c1_w, ctx_conv2_c1_b, ctx_conv2_c1_a,
         ctx_conv2_c2_w, ctx_conv2_c2_b, ctx_conv2_c2_a),
        (ctx_conv3_c1_w, ctx_conv3_c1_b, ctx_conv3_c1_a,
         ctx_conv3_c2_w, ctx_conv3_c2_b, ctx_conv3_c2_a),
        (ctx_conv4_c1_w, ctx_conv4_c1_b, ctx_conv4_c1_a,
         ctx_conv4_c2_w, ctx_conv4_c2_b, ctx_conv4_c2_a),
    ]
    xb = _ring(jnp.concatenate([img0, img1], axis=0))
    fb = jnp.concatenate([flow[..., :2], flow[..., 2:4]], axis=0)
    xb = _block_f([xb], *ctx[0])
    xb = _block_f([xb], *ctx[1])
    feats = []
    for lvl in range(2, 5):
        fb = _resize_half(fb) * 0.5
        feats.append(_warp(_valid(xb), fb, 1))
        xb = _block_f([xb], *ctx[lvl])
    fb = _resize_half(fb) * 0.5
    feats.append(_warp(_valid(xb), fb, 1))
    c0 = [fz[:n] for fz in feats]
    c1 = [fz[n:] for fz in feats]

    flow_up = _resize_up2(flow) * 2.0
    wboth = _warp(jnp.concatenate([img0, img1], axis=0),
                  jnp.concatenate([flow_up[..., :2], flow_up[..., 2:4]], axis=0), 4)
    w0, w1 = wboth[:n], wboth[n:]
    x = _block_f([_ring(jnp.concatenate([w0, w1, flow_up.astype(jnp.bfloat16)], -1))],
                 fus_conv0_c1_w, fus_conv0_c1_b, fus_conv0_c1_a,
                 fus_conv0_c2_w, fus_conv0_c2_b, fus_conv0_c2_a)
    s0 = _block_f([x], fus_down0_c1_w, fus_down0_c1_b, fus_down0_c1_a,
                  fus_down0_c2_w, fus_down0_c2_b, fus_down0_c2_a)
    s1 = _block_f([s0, _ring(c0[0]), _ring(c1[0])],
                  fus_down1_c1_w, fus_down1_c1_b, fus_down1_c1_a,
                  fus_down1_c2_w, fus_down1_c2_b, fus_down1_c2_a)
    s2 = _block_f([s1, _ring(c0[1]), _ring(c1[1])],
                  fus_down2_c1_w, fus_down2_c1_b, fus_down2_c1_a,
                  fus_down2_c2_w, fus_down2_c2_b, fus_down2_c2_a)
    s3 = _block_f([s2, _ring(c0[2]), _ring(c1[2])],
                  fus_down3_c1_w, fus_down3_c1_b, fus_down3_c1_a,
                  fus_down3_c2_w, fus_down3_c2_b, fus_down3_c2_a)
    u = _deconv_f([s3, _ring(c0[3]), _ring(c1[3])], fus_up0_w, fus_up0_b, fus_up0_a)
    u = _deconv_f([_ring(_d2s_valid(u, 256)), s2], fus_up1_w, fus_up1_b, fus_up1_a)
    u = _deconv_f([_ring(_d2s_valid(u, 128)), s1], fus_up2_w, fus_up2_b, fus_up2_a)
    u = _deconv_f([_ring(_d2s_valid(u, 64)), s0], fus_up3_w, fus_up3_b, fus_up3_a)
    x = _ring(_d2s_valid(u, 32))
    return _final_f(x, fus_conv_w, fus_conv_b, w0, w1)
